# baseline (speedup 1.0000x reference)
import jax
import jax.numpy as jnp
from jax.experimental import pallas as pl

_N = 10000
_E = 160000
_B = 64
_EPS = 1e-5
_TE = 2000
_TN = 2000


def _affine(sums, cnt, g, b):
    m = sums[0] / cnt
    v = sums[1] / cnt - m * m
    sc = g * jax.lax.rsqrt(v + _EPS)
    return (sc.reshape(1, -1), (b - m * sc).reshape(1, -1))


def _stats_pass(inputs, tile):
    n = inputs[0].shape[0]
    grid = n // tile
    win = sum(a.shape[1] for a in inputs)
    n_in = len(inputs)

    def body(*refs):
        s_ref = refs[-1]
        cat = (refs[0][...] if n_in == 1
               else jnp.concatenate([r[...] for r in refs[:n_in]], axis=1))
        blk = jnp.concatenate(
            [jnp.sum(cat, 0, keepdims=True),
             jnp.sum(cat * cat, 0, keepdims=True),
             jnp.zeros((6, win), jnp.float32)], axis=0)

        @pl.when(pl.program_id(0) == 0)
        def _():
            s_ref[...] = blk

        @pl.when(pl.program_id(0) != 0)
        def _():
            s_ref[...] += blk

    return pl.pallas_call(
        body,
        grid=(grid,),
        in_specs=[pl.BlockSpec((tile, a.shape[1]), lambda i: (i, 0))
                  for a in inputs],
        out_specs=pl.BlockSpec((8, win), lambda i: (0, 0)),
        out_shape=jax.ShapeDtypeStruct((8, win), jnp.float32),
    )(*inputs)


def _mlp_pass(inputs, scale, shift, wt, b, relu, want_sums, head=None,
              tile=_TE):
    n = inputs[0].shape[0]
    grid = n // tile
    win = sum(a.shape[1] for a in inputs)
    wout = wt.shape[1]
    n_in = len(inputs)

    args = list(inputs) + [scale, shift, wt, b.reshape(1, -1)]
    in_specs = ([pl.BlockSpec((tile, a.shape[1]), lambda i: (i, 0))
                 for a in inputs] +
                [pl.BlockSpec((1, win), lambda i: (0, 0)),
                 pl.BlockSpec((1, win), lambda i: (0, 0)),
                 pl.BlockSpec((win, wout), lambda i: (0, 0)),
                 pl.BlockSpec((1, wout), lambda i: (0, 0))])
    if head is not None:
        wh = head[0].shape[1]
        args += [head[0], head[1].reshape(1, -1)]
        in_specs += [pl.BlockSpec((wout, wh), lambda i: (0, 0)),
                     pl.BlockSpec((1, wh), lambda i: (0, 0))]

    out_shape = [jax.ShapeDtypeStruct((n, wout), jnp.float32)]
    out_specs = [pl.BlockSpec((tile, wout), lambda i: (i, 0))]
    if want_sums:
        out_shape.append(jax.ShapeDtypeStruct((8, wout), jnp.float32))
        out_specs.append(pl.BlockSpec((8, wout), lambda i: (0, 0)))
    if head is not None:
        out_shape.append(jax.ShapeDtypeStruct((n, wh), jnp.float32))
        out_specs.append(pl.BlockSpec((tile, wh), lambda i: (i, 0)))

    def body(*refs):
        in_refs = refs[:n_in]
        sc_ref, sh_ref, w_ref, b_ref = refs[n_in:n_in + 4]
        k = n_in + 4
        if head is not None:
            hw_ref, hb_ref = refs[k], refs[k + 1]
            k += 2
        y_ref = refs[k]
        k += 1
        if want_sums:
            s_ref = refs[k]
            k += 1
        cat = (in_refs[0][...] if n_in == 1
               else jnp.concatenate([r[...] for r in in_refs], axis=1))
        xn = cat * sc_ref[...] + sh_ref[...]
        y = jnp.dot(xn.astype(jnp.bfloat16), w_ref[...].astype(jnp.bfloat16),
                    preferred_element_type=jnp.float32) + b_ref[...]
        if relu:
            y = jnp.maximum(y, 0.0)
        y_ref[...] = y
        if want_sums:
            blk = jnp.concatenate(
                [jnp.sum(y, 0, keepdims=True),
                 jnp.sum(y * y, 0, keepdims=True),
                 jnp.zeros((6, wout), jnp.float32)], axis=0)

            @pl.when(pl.program_id(0) == 0)
            def _():
                s_ref[...] = blk

            @pl.when(pl.program_id(0) != 0)
            def _():
                s_ref[...] += blk
        if head is not None:
            h_ref = refs[k]
            h_ref[...] = jnp.dot(y.astype(jnp.bfloat16),
                                 hw_ref[...].astype(jnp.bfloat16),
                                 preferred_element_type=jnp.float32
                                 ) + hb_ref[...]

    res = pl.pallas_call(
        body, grid=(grid,), in_specs=in_specs,
        out_specs=out_specs, out_shape=out_shape)(*args)
    return res


def _msg_pass(h2, xr, s2c, s2s, nsc, nsh, w2t, b2):
    grid = _E // _TE

    def body(h2_ref, xr_ref, s2c_r, s2s_r, nsc_r, nsh_r, w_ref, b_ref,
             out_ref):
        h2n = h2_ref[...] * s2c_r[...] + s2s_r[...]
        xnr = xr_ref[...] * nsc_r[...] + nsh_r[...]
        w = jnp.dot(h2n.astype(jnp.bfloat16),
                    w_ref[...].astype(jnp.bfloat16),
                    preferred_element_type=jnp.float32) + b_ref[...]
        xb = xnr.astype(jnp.bfloat16).astype(jnp.float32)
        wb = w.astype(jnp.bfloat16).astype(jnp.float32)
        acc = xb[:, 0:1] * wb[:, 0:16]
        for i in range(1, 16):
            acc = acc + xb[:, i:i + 1] * wb[:, i * 16:(i + 1) * 16]
        out_ref[...] = acc

    vec = lambda: pl.BlockSpec((1, 16), lambda i: (0, 0))
    return pl.pallas_call(
        body, grid=(grid,),
        in_specs=[pl.BlockSpec((_TE, 16), lambda i: (i, 0)),
                  pl.BlockSpec((_TE, 16), lambda i: (i, 0)),
                  vec(), vec(), vec(), vec(),
                  pl.BlockSpec((16, 256), lambda i: (0, 0)),
                  pl.BlockSpec((1, 256), lambda i: (0, 0))],
        out_specs=pl.BlockSpec((_TE, 16), lambda i: (i, 0)),
        out_shape=jax.ShapeDtypeStruct((_E, 16), jnp.float32))(
            h2, xr, s2c, s2s, nsc, nsh, w2t, b2.reshape(1, -1))


def _node_pass(agg, x, nsc, nsh, root, bias, xb2d, head=None):
    grid = _N // _TN

    args = [agg, x, nsc, nsh, root, bias.reshape(1, -1), xb2d]
    in_specs = [pl.BlockSpec((_TN, 16), lambda i: (i, 0)),
                pl.BlockSpec((_TN, 16), lambda i: (i, 0)),
                pl.BlockSpec((1, 16), lambda i: (0, 0)),
                pl.BlockSpec((1, 16), lambda i: (0, 0)),
                pl.BlockSpec((16, 16), lambda i: (0, 0)),
                pl.BlockSpec((1, 16), lambda i: (0, 0)),
                pl.BlockSpec((_TN, 1), lambda i: (i, 0))]
    if head is not None:
        wh = head[0].shape[1]
        args += [head[0], head[1].reshape(1, -1)]
        in_specs += [pl.BlockSpec((16, wh), lambda i: (0, 0)),
                     pl.BlockSpec((1, wh), lambda i: (0, 0))]

    out_shape = [jax.ShapeDtypeStruct((_N, 16), jnp.float32),
                 jax.ShapeDtypeStruct((8, 16), jnp.float32),
                 jax.ShapeDtypeStruct((_B, 16), jnp.float32),
                 jax.ShapeDtypeStruct((8, _B), jnp.float32)]
    out_specs = [pl.BlockSpec((_TN, 16), lambda i: (i, 0)),
                 pl.BlockSpec((8, 16), lambda i: (0, 0)),
                 pl.BlockSpec((_B, 16), lambda i: (0, 0)),
                 pl.BlockSpec((8, _B), lambda i: (0, 0))]
    if head is not None:
        out_shape.append(jax.ShapeDtypeStruct((_N, wh), jnp.float32))
        out_specs.append(pl.BlockSpec((_TN, wh), lambda i: (i, 0)))

    def body(*refs):
        agg_r, x_r, nsc_r, nsh_r, root_r, bias_r, xb_r = refs[:7]
        k = 7
        if head is not None:
            hw_r, hb_r = refs[k], refs[k + 1]
            k += 2
        xo_r, ns_r, gs_r, gc_r = refs[k:k + 4]
        k += 4
        xn = x_r[...] * nsc_r[...] + nsh_r[...]
        y = jnp.maximum(
            agg_r[...] + jnp.dot(xn.astype(jnp.bfloat16),
                                 root_r[...].astype(jnp.bfloat16),
                                 preferred_element_type=jnp.float32)
            + bias_r[...], 0.0)
        xo_r[...] = y
        nblk = jnp.concatenate(
            [jnp.sum(y, 0, keepdims=True),
             jnp.sum(y * y, 0, keepdims=True),
             jnp.zeros((6, 16), jnp.float32)], axis=0)
        onehot = (xb_r[...] == jax.lax.broadcasted_iota(
            jnp.int32, (1, _B), 1)).astype(jnp.float32)
        gs = jax.lax.dot_general(onehot, y, (((0,), (0,)), ((), ())),
                                 preferred_element_type=jnp.float32, precision=jax.lax.Precision.HIGHEST)
        gc = jnp.concatenate([jnp.sum(onehot, 0, keepdims=True),
                              jnp.zeros((7, _B), jnp.float32)], axis=0)

        @pl.when(pl.program_id(0) == 0)
        def _():
            ns_r[...] = nblk
            gs_r[...] = gs
            gc_r[...] = gc

        @pl.when(pl.program_id(0) != 0)
        def _():
            ns_r[...] += nblk
            gs_r[...] += gs
            gc_r[...] += gc

        if head is not None:
            refs[k][...] = jnp.dot(
                y.astype(jnp.bfloat16), hw_r[...].astype(jnp.bfloat16),
                preferred_element_type=jnp.float32) + hb_r[...]

    return pl.pallas_call(
        body, grid=(grid,), in_specs=in_specs,
        out_specs=out_specs, out_shape=out_shape)(*args)


def _glob_pass(u, mean, p, head=None):
    wu = u.shape[1]
    wcat = wu + 16
    mats = [p['W0'].T, p['W1'].T, p['W2'].T]
    vecs = [p['g0'], p['be0'], p['b0'], p['g1'], p['be1'], p['b1'],
            p['g2'], p['be2'], p['b2']]
    args = [u, mean] + mats + [v.reshape(1, -1) for v in vecs]
    in_specs = [pl.BlockSpec((_B, wu), lambda: (0, 0)),
                pl.BlockSpec((_B, 16), lambda: (0, 0))]
    in_specs += [pl.BlockSpec(m.shape, lambda: (0, 0)) for m in mats]
    in_specs += [pl.BlockSpec((1, v.shape[0]), lambda: (0, 0)) for v in vecs]
    wout = p['W2'].shape[0]
    out_shape = [jax.ShapeDtypeStruct((_B, wout), jnp.float32)]
    out_specs = [pl.BlockSpec((_B, wout), lambda: (0, 0))]
    if head is not None:
        wh = head[0].shape[1]
        args += [head[0], head[1].reshape(1, -1)]
        in_specs += [pl.BlockSpec((wout, wh), lambda: (0, 0)),
                     pl.BlockSpec((1, wh), lambda: (0, 0))]
        out_shape.append(jax.ShapeDtypeStruct((_B, wh), jnp.float32))
        out_specs.append(pl.BlockSpec((_B, wh), lambda: (0, 0)))

    def body(*refs):
        u_r, m_r = refs[0], refs[1]
        w0_r, w1_r, w2_r = refs[2], refs[3], refs[4]
        (g0, be0, b0, g1, be1, b1, g2, be2, b2) = refs[5:14]
        k = 14
        if head is not None:
            hw_r, hb_r = refs[k], refs[k + 1]
            k += 2
        out_r = refs[k]
        k += 1

        def bn(xx, g_r, be_r):
            mm = jnp.mean(xx, 0, keepdims=True)
            vv = jnp.mean((xx - mm) ** 2, 0, keepdims=True)
            return (xx - mm) * jax.lax.rsqrt(vv + _EPS) * g_r[...] + be_r[...]

        def bdot(a, b):
            return jnp.dot(a.astype(jnp.bfloat16), b.astype(jnp.bfloat16),
                           preferred_element_type=jnp.float32)

        xx = jnp.concatenate([u_r[...], m_r[...]], axis=1)
        xx = jnp.maximum(bdot(bn(xx, g0, be0), w0_r[...]) + b0[...], 0.0)
        xx = jnp.maximum(bdot(bn(xx, g1, be1), w1_r[...]) + b1[...], 0.0)
        xx = bdot(bn(xx, g2, be2), w2_r[...]) + b2[...]
        out_r[...] = xx
        if head is not None:
            refs[k][...] = bdot(xx, hw_r[...]) + hb_r[...]

    return pl.pallas_call(body, in_specs=in_specs, out_specs=out_specs,
                          out_shape=out_shape)(*args)


def kernel(node_features, edge_indices, edge_features, global_features,
           xbatch, params):
    x = node_features.astype(jnp.float32)
    e = edge_features.astype(jnp.float32)
    u = global_features.astype(jnp.float32)
    row = edge_indices[0]
    col = edge_indices[1]
    xb2d = xbatch.astype(jnp.int32).reshape(_N, 1)

    e_sums = _stats_pass([e], _TE)
    x_sums = _stats_pass([x], _TN)
    xr_raw = jnp.take(x, row, axis=0)

    node_pred = edge_pred = glob_pred = None
    for i in range(3):
        lay = params['mp%d' % i]
        last = (i == 2)
        nsc, nsh = _affine(x_sums, float(_N), lay['bn_g'], lay['bn_b'])
        em = lay['edge_mlp']
        sc0, sh0 = _affine(e_sums, float(_E), em['g0'], em['be0'])
        h1, h1_sums = _mlp_pass([e], sc0, sh0, em['W0'].T, em['b0'],
                                relu=True, want_sums=True)
        sc1, sh1 = _affine(h1_sums, float(_E), em['g1'], em['be1'])
        h2, h2_sums = _mlp_pass([h1], sc1, sh1, em['W1'].T, em['b1'],
                                relu=True, want_sums=True)
        sc2, sh2 = _affine(h2_sums, float(_E), em['g2'], em['be2'])
        msg = _msg_pass(h2, xr_raw, sc2, sh2, nsc, nsh, em['W2'].T,
                        em['b2'])
        agg = jax.ops.segment_sum(msg, col, num_segments=_N)

        nh = ((params['node_W'].T, params['node_b']) if last else None)
        nres = _node_pass(agg, x, nsc, nsh, lay['root'], lay['bias'],
                          xb2d, head=nh)
        x_new, x_sums, gsum, gcnt = nres[:4]
        if last:
            node_pred = nres[4]

        xr_raw = jnp.take(x_new, row, axis=0)
        xc_raw = jnp.take(x_new, col, axis=0)
        rc_sums = _stats_pass([xr_raw, xc_raw], _TE)
        cat_sums = jnp.concatenate([rc_sums, e_sums], axis=1)

        el = lay['edge_layer']
        c0, s0 = _affine(cat_sums, float(_E), el['g0'], el['be0'])
        g1v, g1_sums = _mlp_pass([xr_raw, xc_raw, e], c0, s0,
                                 el['W0'].T, el['b0'],
                                 relu=True, want_sums=True)
        c1, s1 = _affine(g1_sums, float(_E), el['g1'], el['be1'])
        g2v, g2_sums = _mlp_pass([g1v], c1, s1, el['W1'].T, el['b1'],
                                 relu=True, want_sums=True)
        c2, s2 = _affine(g2_sums, float(_E), el['g2'], el['be2'])
        eh = ((params['edge_W'].T, params['edge_b']) if last else None)
        eres = _mlp_pass([g2v], c2, s2, el['W2'].T, el['b2'],
                         relu=False, want_sums=True, head=eh)
        e, e_sums = eres[0], eres[1]
        if last:
            edge_pred = eres[2]

        cnt = jnp.maximum(gcnt[0], 1.0)
        mean = gsum / cnt[:, None]
        gh = ((params['glob_W'].T, params['glob_b']) if last else None)
        gres = _glob_pass(u, mean, lay['global'], head=gh)
        u = gres[0]
        if last:
            glob_pred = gres[1]
        x = x_new

    return (node_pred, edge_pred, glob_pred)
